# baseline (speedup 1.0000x reference)
import functools

import jax
import jax.numpy as jnp
from jax import lax
from jax.experimental import pallas as pl
from jax.experimental.pallas import tpu as pltpu
from jax.experimental.pallas import tpu_sc as plsc

N = 10000
E = 320000
D = 128
G = 64
L = 3

C = 80
NW = 32
CPW = E // C // NW
RPT = 624

BLK = 1000
NBLK = N // BLK


def _sc_aggregate(h, eidx, ew3, zeros):
    mesh = plsc.VectorSubcoreMesh(core_axis_name="c", subcore_axis_name="s")

    @functools.partial(
        pl.kernel,
        out_type=jax.ShapeDtypeStruct((2, N, D), jnp.float32),
        mesh=mesh,
        scratch_types=[
            pltpu.VMEM((4, 2, C), jnp.int32),
            pltpu.VMEM((4, 1, C), jnp.float32),
            pltpu.VMEM((C, D), jnp.float32),
            pltpu.VMEM((C, D), jnp.float32),
            pltpu.VMEM((C, D), jnp.float32),
            pltpu.VMEM((C, D), jnp.float32),
            pltpu.VMEM_SHARED((N, D), jnp.float32),
        ] + [pltpu.SemaphoreType.DMA] * 16,
    )
    def k(h_hbm, ed_hbm, ew_hbm, z_hbm, out_hbm, ebuf, wbuf,
          rows0, rows1, rows2, rows3, acc_sh,
          i0, i1, i2, i3, w0, w1, w2, w3, g0, g1, g2, g3,
          s0, s1, s2, s3):
        cc = lax.axis_index("c")
        ss = lax.axis_index("s")
        w = ss * 2 + cc
        rows = (rows0, rows1, rows2, rows3)
        isem = (i0, i1, i2, i3)
        wsem = (w0, w1, w2, w3)
        gsem = (g0, g1, g2, g3)
        ssem = (s0, s1, s2, s3)
        ed_w = ed_hbm.at[w]
        ew_w = ew_hbm.at[w]

        pltpu.sync_copy(z_hbm.at[pl.ds(ss * RPT, RPT)],
                        acc_sh.at[pl.ds(ss * RPT, RPT)])

        @pl.when(ss == 15)
        def _():
            pltpu.sync_copy(z_hbm.at[pl.ds(16 * RPT, N - 16 * RPT)],
                            acc_sh.at[pl.ds(16 * RPT, N - 16 * RPT)])

        plsc.subcore_barrier()

        def issue_idx(b, kk):
            pltpu.async_copy(ed_w.at[kk], ebuf.at[b], isem[b])
            pltpu.async_copy(ew_w.at[kk], wbuf.at[b], wsem[b])

        def iwait(b):
            pltpu.make_async_copy(ed_w.at[0], ebuf.at[b], isem[b]).wait()
            pltpu.make_async_copy(ew_w.at[0], wbuf.at[b], wsem[b]).wait()

        def issue_gather(b):
            pltpu.async_copy(h_hbm.at[ebuf.at[b].at[0]], rows[b], gsem[b])

        def gwait(b):
            pltpu.make_async_copy(h_hbm.at[pl.ds(0, C)], rows[b],
                                  gsem[b]).wait()

        def issue_scat(b):
            pltpu.async_copy(rows[b], acc_sh.at[ebuf.at[b].at[1]], ssem[b],
                             add=True)

        def swait(b):
            pltpu.make_async_copy(h_hbm.at[pl.ds(0, C)], rows[b],
                                  ssem[b]).wait()

        def scale(b):
            wb = wbuf.at[b].at[0]
            rb = rows[b]

            @plsc.parallel_loop(0, C // 16, 1, unroll=2)
            def _(g):
                wrow = wb[pl.ds(g * 16, 16)]
                for l in range(16):
                    wv = jnp.full((16,), wrow[l])
                    for j in range(D // 16):
                        rb[g * 16 + l, pl.ds(j * 16, 16)] = (
                            rb[g * 16 + l, pl.ds(j * 16, 16)] * wv)

        def substep(b, kk, do_swait=True, refill=True, prefetch=True):
            b2 = (b + 2) % 4
            if do_swait:
                swait(b2)
            if refill:
                issue_idx(b2, kk + 2)
            gwait(b)
            scale(b)
            issue_scat(b)
            if prefetch:
                iwait(b2)
                issue_gather(b2)

        issue_idx(0, 0)
        issue_idx(1, 1)
        iwait(0)
        issue_gather(0)
        iwait(1)
        issue_gather(1)
        substep(0, 0, do_swait=False)
        substep(1, 1, do_swait=False)
        substep(2, 2)
        substep(3, 3)

        def body(i, carry):
            kk = 4 * i
            substep(0, kk)
            substep(1, kk + 1)
            substep(2, kk + 2)
            substep(3, kk + 3)
            return carry

        lax.fori_loop(1, CPW // 4 - 1, body, 0)

        substep(0, CPW - 5)
        substep(1, CPW - 4)
        substep(2, CPW - 3)
        substep(3, CPW - 2, refill=False, prefetch=False)
        substep(0, CPW - 1, refill=False, prefetch=False)
        swait(3)
        swait(0)

        plsc.subcore_barrier()
        pltpu.sync_copy(acc_sh.at[pl.ds(ss * RPT, RPT)],
                        out_hbm.at[cc].at[pl.ds(ss * RPT, RPT)])

        @pl.when(ss == 15)
        def _():
            pltpu.sync_copy(acc_sh.at[pl.ds(16 * RPT, N - 16 * RPT)],
                            out_hbm.at[cc].at[pl.ds(16 * RPT, N - 16 * RPT)])

    return k(h, eidx, ew3, zeros)


def _layer_body(h_ref, p0_ref, p1_ref, w1_ref, b1_ref, w2_ref, b2_ref,
                g_ref, be_ref, bt_ref, z_ref, pool_ref, u_scr, s_scr):
    p = pl.program_id(0)
    i = pl.program_id(1)

    @pl.when(jnp.logical_and(p == 0, i == 0))
    def _():
        s_scr[...] = jnp.zeros_like(s_scr)
        pool_ref[...] = jnp.zeros_like(pool_ref)

    @pl.when(p == 0)
    def _():
        z = h_ref[...] + p0_ref[...] + p1_ref[...]
        a = jnp.maximum(
            jnp.dot(z, w1_ref[...], preferred_element_type=jnp.float32)
            + b1_ref[...], 0.0)
        u = jnp.maximum(
            jnp.dot(a, w2_ref[...], preferred_element_type=jnp.float32)
            + b2_ref[...], 0.0)
        u_scr[pl.ds(i * BLK, BLK), :] = u
        su = jnp.sum(u, axis=0, keepdims=True)
        sq = jnp.sum(u * u, axis=0, keepdims=True)
        s_scr[...] += jnp.concatenate([su, sq], axis=0)
        z_ref[...] = u

    @pl.when(p == 1)
    def _():
        u = u_scr[pl.ds(i * BLK, BLK), :]
        mean = s_scr[0:1, :] * (1.0 / N)
        var = s_scr[1:2, :] * (1.0 / N) - mean * mean
        scl = lax.rsqrt(var + 1e-5) * g_ref[...]
        z = (u - mean) * scl + be_ref[...]
        z_ref[...] = z
        b = bt_ref[0, 0, :]
        gi = lax.broadcasted_iota(jnp.int32, (BLK, G), 1)
        onehot = (b[:, None] == gi).astype(jnp.bfloat16)
        zhi = z.astype(jnp.bfloat16)
        zlo = (z - zhi.astype(jnp.float32)).astype(jnp.bfloat16)
        dn = (((0,), (0,)), ((), ()))
        pool_ref[...] += (
            lax.dot_general(onehot, zhi, dn,
                            preferred_element_type=jnp.float32)
            + lax.dot_general(onehot, zlo, dn,
                              preferred_element_type=jnp.float32))


def _tc_layer(h, p0, p1, W1, b1, W2, b2, gamma, beta, batch3d):
    return pl.pallas_call(
        _layer_body,
        grid=(2, NBLK),
        in_specs=[
            pl.BlockSpec((BLK, D), lambda p, i: (i, 0)),
            pl.BlockSpec((BLK, D), lambda p, i: (i, 0)),
            pl.BlockSpec((BLK, D), lambda p, i: (i, 0)),
            pl.BlockSpec((D, D), lambda p, i: (0, 0)),
            pl.BlockSpec((1, D), lambda p, i: (0, 0)),
            pl.BlockSpec((D, D), lambda p, i: (0, 0)),
            pl.BlockSpec((1, D), lambda p, i: (0, 0)),
            pl.BlockSpec((1, D), lambda p, i: (0, 0)),
            pl.BlockSpec((1, D), lambda p, i: (0, 0)),
            pl.BlockSpec((1, 1, BLK), lambda p, i: (i, 0, 0)),
        ],
        out_specs=[
            pl.BlockSpec((BLK, D), lambda p, i: (i, 0)),
            pl.BlockSpec((G, D), lambda p, i: (0, 0)),
        ],
        out_shape=[
            jax.ShapeDtypeStruct((N, D), jnp.float32),
            jax.ShapeDtypeStruct((G, D), jnp.float32),
        ],
        scratch_shapes=[
            pltpu.VMEM((N, D), jnp.float32),
            pltpu.VMEM((2, D), jnp.float32),
        ],
    )(h, p0, p1, W1, b1.reshape(1, D), W2, b2.reshape(1, D),
      gamma.reshape(1, D), beta.reshape(1, D), batch3d)


def kernel(x, edge_index, batch, edge_weight,
           W1_0, b1_0, W2_0, b2_0, gamma_0, beta_0,
           W1_1, b1_1, W2_1, b2_1, gamma_1, beta_1,
           W1_2, b1_2, W2_2, b2_2, gamma_2, beta_2):
    params = [
        (W1_0, b1_0, W2_0, b2_0, gamma_0, beta_0),
        (W1_1, b1_1, W2_1, b2_1, gamma_1, beta_1),
        (W1_2, b1_2, W2_2, b2_2, gamma_2, beta_2),
    ]
    src4 = edge_index[0].reshape(NW, CPW, 1, C)
    dst4 = edge_index[1].reshape(NW, CPW, 1, C)
    eidx = jnp.concatenate([src4, dst4], axis=2)
    ew3 = edge_weight.reshape(NW, CPW, 1, C)
    zeros = jnp.zeros((N, D), jnp.float32)
    batch3d = batch.reshape(NBLK, 1, BLK)

    h = x
    zs, pools = [], []
    for (W1, b1, W2, b2, ga, be) in params:
        parts = _sc_aggregate(h, eidx, ew3, zeros)
        z, pool = _tc_layer(h, parts[0], parts[1], W1, b1, W2, b2,
                            ga, be, batch3d)
        zs.append(z)
        pools.append(pool)
        h = z
    x_g = jnp.concatenate(pools, axis=1)
    x_all = jnp.concatenate(zs, axis=1)
    return (x_g, x_all)


# speedup vs baseline: 1.1523x; 1.1523x over previous
import functools

import jax
import jax.numpy as jnp
from jax import lax
from jax.experimental import pallas as pl
from jax.experimental.pallas import tpu as pltpu
from jax.experimental.pallas import tpu_sc as plsc

N = 10000
E = 320000
D = 128
G = 64
L = 3

C = 80
NW = 32
CPW = E // C // NW
RPT = 624

BLK = 2000
NBLK = N // BLK


def _sc_aggregate(h, eidx, ew3, zeros):
    mesh = plsc.VectorSubcoreMesh(core_axis_name="c", subcore_axis_name="s")

    @functools.partial(
        pl.kernel,
        out_type=jax.ShapeDtypeStruct((2, N, D), jnp.float32),
        mesh=mesh,
        scratch_types=[
            pltpu.VMEM((4, 2, C), jnp.int32),
            pltpu.VMEM((4, 1, C), jnp.float32),
            pltpu.VMEM((C, D), jnp.float32),
            pltpu.VMEM((C, D), jnp.float32),
            pltpu.VMEM((C, D), jnp.float32),
            pltpu.VMEM((C, D), jnp.float32),
            pltpu.VMEM_SHARED((N, D), jnp.float32),
        ] + [pltpu.SemaphoreType.DMA] * 16,
    )
    def k(h_hbm, ed_hbm, ew_hbm, z_hbm, out_hbm, ebuf, wbuf,
          rows0, rows1, rows2, rows3, acc_sh,
          i0, i1, i2, i3, w0, w1, w2, w3, g0, g1, g2, g3,
          s0, s1, s2, s3):
        cc = lax.axis_index("c")
        ss = lax.axis_index("s")
        w = ss * 2 + cc
        rows = (rows0, rows1, rows2, rows3)
        isem = (i0, i1, i2, i3)
        wsem = (w0, w1, w2, w3)
        gsem = (g0, g1, g2, g3)
        ssem = (s0, s1, s2, s3)
        ed_w = ed_hbm.at[w]
        ew_w = ew_hbm.at[w]

        pltpu.sync_copy(z_hbm.at[pl.ds(ss * RPT, RPT)],
                        acc_sh.at[pl.ds(ss * RPT, RPT)])

        @pl.when(ss == 15)
        def _():
            pltpu.sync_copy(z_hbm.at[pl.ds(16 * RPT, N - 16 * RPT)],
                            acc_sh.at[pl.ds(16 * RPT, N - 16 * RPT)])

        plsc.subcore_barrier()

        def issue_idx(b, kk):
            pltpu.async_copy(ed_w.at[kk], ebuf.at[b], isem[b])
            pltpu.async_copy(ew_w.at[kk], wbuf.at[b], wsem[b])

        def iwait(b):
            pltpu.make_async_copy(ed_w.at[0], ebuf.at[b], isem[b]).wait()
            pltpu.make_async_copy(ew_w.at[0], wbuf.at[b], wsem[b]).wait()

        def issue_gather(b):
            pltpu.async_copy(h_hbm.at[ebuf.at[b].at[0]], rows[b], gsem[b])

        def gwait(b):
            pltpu.make_async_copy(h_hbm.at[pl.ds(0, C)], rows[b],
                                  gsem[b]).wait()

        def issue_scat(b):
            pltpu.async_copy(rows[b], acc_sh.at[ebuf.at[b].at[1]], ssem[b],
                             add=True)

        def swait(b):
            pltpu.make_async_copy(h_hbm.at[pl.ds(0, C)], rows[b],
                                  ssem[b]).wait()

        def scale(b):
            wb = wbuf.at[b].at[0]
            rb = rows[b]

            def group_body(g, carry):
                wrow = wb[pl.ds(g * 16, 16)]
                for l in range(16):
                    wv = jnp.full((16,), wrow[l])
                    for j in range(D // 16):
                        rb[g * 16 + l, pl.ds(j * 16, 16)] = (
                            rb[g * 16 + l, pl.ds(j * 16, 16)] * wv)
                return carry

            lax.fori_loop(0, C // 16, group_body, 0)

        def substep(b, kk, do_swait=True, refill=True, prefetch=True):
            b2 = (b + 2) % 4
            if do_swait:
                swait(b2)
            if refill:
                issue_idx(b2, kk + 2)
            gwait(b)
            scale(b)
            issue_scat(b)
            if prefetch:
                iwait(b2)
                issue_gather(b2)

        issue_idx(0, 0)
        issue_idx(1, 1)
        iwait(0)
        issue_gather(0)
        iwait(1)
        issue_gather(1)
        substep(0, 0, do_swait=False)
        substep(1, 1, do_swait=False)
        substep(2, 2)
        substep(3, 3)

        def body(i, carry):
            kk = 4 * i
            substep(0, kk)
            substep(1, kk + 1)
            substep(2, kk + 2)
            substep(3, kk + 3)
            return carry

        lax.fori_loop(1, CPW // 4 - 1, body, 0)

        substep(0, CPW - 5)
        substep(1, CPW - 4)
        substep(2, CPW - 3)
        substep(3, CPW - 2, refill=False, prefetch=False)
        substep(0, CPW - 1, refill=False, prefetch=False)
        swait(3)
        swait(0)

        plsc.subcore_barrier()
        pltpu.sync_copy(acc_sh.at[pl.ds(ss * RPT, RPT)],
                        out_hbm.at[cc].at[pl.ds(ss * RPT, RPT)])

        @pl.when(ss == 15)
        def _():
            pltpu.sync_copy(acc_sh.at[pl.ds(16 * RPT, N - 16 * RPT)],
                            out_hbm.at[cc].at[pl.ds(16 * RPT, N - 16 * RPT)])

    return k(h, eidx, ew3, zeros)


def _layer_body(h_ref, p0_ref, p1_ref, w1_ref, b1_ref, w2_ref, b2_ref,
                g_ref, be_ref, bt_ref, z_ref, pool_ref, u_scr, s_scr):
    p = pl.program_id(0)
    i = pl.program_id(1)

    @pl.when(jnp.logical_and(p == 0, i == 0))
    def _():
        s_scr[...] = jnp.zeros_like(s_scr)
        pool_ref[...] = jnp.zeros_like(pool_ref)

    @pl.when(p == 0)
    def _():
        z = h_ref[...] + p0_ref[...] + p1_ref[...]
        a = jnp.maximum(
            jnp.dot(z, w1_ref[...], preferred_element_type=jnp.float32)
            + b1_ref[...], 0.0)
        u = jnp.maximum(
            jnp.dot(a, w2_ref[...], preferred_element_type=jnp.float32)
            + b2_ref[...], 0.0)
        u_scr[pl.ds(i * BLK, BLK), :] = u
        su = jnp.sum(u, axis=0, keepdims=True)
        sq = jnp.sum(u * u, axis=0, keepdims=True)
        s_scr[...] += jnp.concatenate([su, sq], axis=0)
        z_ref[...] = u

    @pl.when(p == 1)
    def _():
        u = u_scr[pl.ds(i * BLK, BLK), :]
        mean = s_scr[0:1, :] * (1.0 / N)
        var = s_scr[1:2, :] * (1.0 / N) - mean * mean
        scl = lax.rsqrt(var + 1e-5) * g_ref[...]
        z = (u - mean) * scl + be_ref[...]
        z_ref[...] = z
        b = bt_ref[0, 0, :]
        gi = lax.broadcasted_iota(jnp.int32, (BLK, G), 1)
        onehot = (b[:, None] == gi).astype(jnp.bfloat16)
        zhi = z.astype(jnp.bfloat16)
        zlo = (z - zhi.astype(jnp.float32)).astype(jnp.bfloat16)
        dn = (((0,), (0,)), ((), ()))
        pool_ref[...] += (
            lax.dot_general(onehot, zhi, dn,
                            preferred_element_type=jnp.float32)
            + lax.dot_general(onehot, zlo, dn,
                              preferred_element_type=jnp.float32))


def _tc_layer(h, p0, p1, W1, b1, W2, b2, gamma, beta, batch3d):
    return pl.pallas_call(
        _layer_body,
        grid=(2, NBLK),
        in_specs=[
            pl.BlockSpec((BLK, D), lambda p, i: (i, 0)),
            pl.BlockSpec((BLK, D), lambda p, i: (i, 0)),
            pl.BlockSpec((BLK, D), lambda p, i: (i, 0)),
            pl.BlockSpec((D, D), lambda p, i: (0, 0)),
            pl.BlockSpec((1, D), lambda p, i: (0, 0)),
            pl.BlockSpec((D, D), lambda p, i: (0, 0)),
            pl.BlockSpec((1, D), lambda p, i: (0, 0)),
            pl.BlockSpec((1, D), lambda p, i: (0, 0)),
            pl.BlockSpec((1, D), lambda p, i: (0, 0)),
            pl.BlockSpec((1, 1, BLK), lambda p, i: (i, 0, 0)),
        ],
        out_specs=[
            pl.BlockSpec((BLK, D), lambda p, i: (i, 0)),
            pl.BlockSpec((G, D), lambda p, i: (0, 0)),
        ],
        out_shape=[
            jax.ShapeDtypeStruct((N, D), jnp.float32),
            jax.ShapeDtypeStruct((G, D), jnp.float32),
        ],
        scratch_shapes=[
            pltpu.VMEM((N, D), jnp.float32),
            pltpu.VMEM((2, D), jnp.float32),
        ],
    )(h, p0, p1, W1, b1.reshape(1, D), W2, b2.reshape(1, D),
      gamma.reshape(1, D), beta.reshape(1, D), batch3d)


def kernel(x, edge_index, batch, edge_weight,
           W1_0, b1_0, W2_0, b2_0, gamma_0, beta_0,
           W1_1, b1_1, W2_1, b2_1, gamma_1, beta_1,
           W1_2, b1_2, W2_2, b2_2, gamma_2, beta_2):
    params = [
        (W1_0, b1_0, W2_0, b2_0, gamma_0, beta_0),
        (W1_1, b1_1, W2_1, b2_1, gamma_1, beta_1),
        (W1_2, b1_2, W2_2, b2_2, gamma_2, beta_2),
    ]
    src4 = edge_index[0].reshape(NW, CPW, 1, C)
    dst4 = edge_index[1].reshape(NW, CPW, 1, C)
    eidx = jnp.concatenate([src4, dst4], axis=2)
    ew3 = edge_weight.reshape(NW, CPW, 1, C)
    zeros = jnp.zeros((N, D), jnp.float32)
    batch3d = batch.reshape(NBLK, 1, BLK)

    h = x
    zs, pools = [], []
    for (W1, b1, W2, b2, ga, be) in params:
        parts = _sc_aggregate(h, eidx, ew3, zeros)
        z, pool = _tc_layer(h, parts[0], parts[1], W1, b1, W2, b2,
                            ga, be, batch3d)
        zs.append(z)
        pools.append(pool)
        h = z
    x_g = jnp.concatenate(pools, axis=1)
    x_all = jnp.concatenate(zs, axis=1)
    return (x_g, x_all)
